# baseline (speedup 1.0000x reference)
import functools

import jax
import jax.numpy as jnp
from jax import lax
from jax.experimental import pallas as pl
from jax.experimental.pallas import tpu as pltpu
from jax.experimental.pallas import tpu_sc as plsc

B, S, D = 16, 2048, 128
L = 16

_info = plsc.get_sparse_core_info()
NC, NS = _info.num_cores, _info.num_subcores
NW = NC * NS
SW = S // NW


def _embed_body(x_hbm, tok_hbm, pos_hbm, out_hbm,
                idx_v, pos_v, buf0, buf1, g0, g1, o0, o1):
    wid = lax.axis_index("s") * NC + lax.axis_index("c")
    s0 = wid * SW

    for b in range(B):
        pltpu.sync_copy(x_hbm.at[pl.ds(b * S + s0, SW)], idx_v.at[b])
    pltpu.sync_copy(pos_hbm.at[pl.ds(s0, SW)], pos_v)

    bufs = (buf0, buf1)
    gsems = (g0, g1)
    osems = (o0, o1)

    def start_gather(b):
        return pltpu.async_copy(tok_hbm.at[idx_v.at[b]], bufs[b % 2], gsems[b % 2])

    gcp = start_gather(0)
    ocp = [None, None]
    for b in range(B):
        cur = b % 2
        buf = bufs[cur]
        this_g = gcp
        if b + 1 < B:
            nxt = (b + 1) % 2
            if ocp[nxt] is not None:
                ocp[nxt].wait()
            gcp = start_gather(b + 1)
        this_g.wait()

        @plsc.parallel_loop(0, SW, unroll=2)
        def add_row(r, buf=buf):
            for c in range(D // L):
                sl = pl.ds(c * L, L)
                buf[r, sl] = buf[r, sl] + pos_v[r, sl]

        ocp[cur] = pltpu.async_copy(buf, out_hbm.at[b, pl.ds(s0, SW)], osems[cur])

    ocp[0].wait()
    ocp[1].wait()


_embed = functools.partial(
    pl.kernel,
    out_type=jax.ShapeDtypeStruct((B, S, D), jnp.float32),
    mesh=plsc.VectorSubcoreMesh(core_axis_name="c", subcore_axis_name="s"),
    scratch_types=[
        pltpu.VMEM((B, SW), jnp.int32),
        pltpu.VMEM((SW, D), jnp.float32),
        pltpu.VMEM((SW, D), jnp.float32),
        pltpu.VMEM((SW, D), jnp.float32),
        pltpu.SemaphoreType.DMA,
        pltpu.SemaphoreType.DMA,
        pltpu.SemaphoreType.DMA,
        pltpu.SemaphoreType.DMA,
    ],
)(_embed_body)


def kernel(x, token_embd, pos_embd):
    return _embed(x.astype(jnp.int32).reshape(B * S), token_embd, pos_embd)


# speedup vs baseline: 1.0804x; 1.0804x over previous
import functools

import jax
import jax.numpy as jnp
from jax import lax
from jax.experimental import pallas as pl
from jax.experimental.pallas import tpu as pltpu
from jax.experimental.pallas import tpu_sc as plsc

B, S, D = 16, 2048, 128
L = 16

_info = plsc.get_sparse_core_info()
NC, NS = _info.num_cores, _info.num_subcores
NW = NC * NS
SW = S // NW


def _embed_body(x_hbm, tok_hbm, pos_hbm, out_hbm,
                idx_v, pos_v, buf0, buf1, g0, g1, o0, o1):
    wid = lax.axis_index("s") * NC + lax.axis_index("c")
    s0 = wid * SW

    for b in range(B):
        pltpu.sync_copy(x_hbm.at[pl.ds(b * S + s0, SW)], idx_v.at[b])
    pltpu.sync_copy(pos_hbm.at[pl.ds(s0, SW)], pos_v)

    bufs = (buf0, buf1)
    gsems = (g0, g1)
    osems = (o0, o1)

    def start_gather(b):
        return pltpu.async_copy(tok_hbm.at[idx_v.at[b]], bufs[b % 2], gsems[b % 2])

    gcp = start_gather(0)
    ocp = [None, None]
    for b in range(B):
        cur = b % 2
        buf = bufs[cur]
        this_g = gcp
        if b + 1 < B:
            nxt = (b + 1) % 2
            if ocp[nxt] is not None:
                ocp[nxt].wait()
            gcp = start_gather(b + 1)
        this_g.wait()

        def add_row(r, carry, buf=buf):
            for c in range(D // L):
                sl = pl.ds(c * L, L)
                buf[r, sl] = buf[r, sl] + pos_v[r, sl]
            return carry

        lax.fori_loop(0, SW, add_row, 0)

        ocp[cur] = pltpu.async_copy(buf, out_hbm.at[b, pl.ds(s0, SW)], osems[cur])

    ocp[0].wait()
    ocp[1].wait()


_embed = functools.partial(
    pl.kernel,
    out_type=jax.ShapeDtypeStruct((B, S, D), jnp.float32),
    mesh=plsc.VectorSubcoreMesh(core_axis_name="c", subcore_axis_name="s"),
    scratch_types=[
        pltpu.VMEM((B, SW), jnp.int32),
        pltpu.VMEM((SW, D), jnp.float32),
        pltpu.VMEM((SW, D), jnp.float32),
        pltpu.VMEM((SW, D), jnp.float32),
        pltpu.SemaphoreType.DMA,
        pltpu.SemaphoreType.DMA,
        pltpu.SemaphoreType.DMA,
        pltpu.SemaphoreType.DMA,
    ],
)(_embed_body)


def kernel(x, token_embd, pos_embd):
    return _embed(x.astype(jnp.int32).reshape(B * S), token_embd, pos_embd)


# speedup vs baseline: 1.1714x; 1.0843x over previous
import functools

import jax
import jax.numpy as jnp
from jax import lax
from jax.experimental import pallas as pl
from jax.experimental.pallas import tpu as pltpu
from jax.experimental.pallas import tpu_sc as plsc

B, S, D = 16, 2048, 128
L = 16
BP = 2

_info = plsc.get_sparse_core_info()
NC, NS = _info.num_cores, _info.num_subcores
NW = NC * NS
SW = S // NW
NG = B // BP


def _embed_body(x_hbm, tok_hbm, pos_hbm, out_hbm,
                idx_v, pos_v, buf0, buf1, g0, g1, o0, o1):
    wid = lax.axis_index("s") * NC + lax.axis_index("c")
    s0 = wid * SW

    for b in range(B):
        k, h = b // BP, b % BP
        pltpu.sync_copy(x_hbm.at[pl.ds(b * S + s0, SW)],
                        idx_v.at[pl.ds((k * BP + h) * SW, SW)])
    pltpu.sync_copy(pos_hbm.at[pl.ds(s0, SW)], pos_v)

    bufs = (buf0, buf1)
    gsems = (g0, g1)
    osems = (o0, o1)

    def start_gather(k):
        return pltpu.async_copy(tok_hbm.at[idx_v.at[pl.ds(k * BP * SW, BP * SW)]],
                                bufs[k % 2], gsems[k % 2])

    gcp = start_gather(0)
    ocp = [[], []]
    for k in range(NG):
        cur = k % 2
        buf = bufs[cur]
        this_g = gcp
        if k + 1 < NG:
            nxt = (k + 1) % 2
            for cp in ocp[nxt]:
                cp.wait()
            ocp[nxt] = []
            gcp = start_gather(k + 1)
        this_g.wait()

        def add_row(r, carry, buf=buf):
            for c in range(D // L):
                sl = pl.ds(c * L, L)
                p = pos_v[r, sl]
                for h in range(BP):
                    buf[h * SW + r, sl] = buf[h * SW + r, sl] + p
            return carry

        lax.fori_loop(0, SW, add_row, 0)

        ocp[cur] = [
            pltpu.async_copy(buf.at[pl.ds(h * SW, SW)],
                             out_hbm.at[k * BP + h, pl.ds(s0, SW)], osems[cur])
            for h in range(BP)
        ]

    for cps in ocp:
        for cp in cps:
            cp.wait()


_embed = functools.partial(
    pl.kernel,
    out_type=jax.ShapeDtypeStruct((B, S, D), jnp.float32),
    mesh=plsc.VectorSubcoreMesh(core_axis_name="c", subcore_axis_name="s"),
    scratch_types=[
        pltpu.VMEM((B * SW,), jnp.int32),
        pltpu.VMEM((SW, D), jnp.float32),
        pltpu.VMEM((BP * SW, D), jnp.float32),
        pltpu.VMEM((BP * SW, D), jnp.float32),
        pltpu.SemaphoreType.DMA,
        pltpu.SemaphoreType.DMA,
        pltpu.SemaphoreType.DMA,
        pltpu.SemaphoreType.DMA,
    ],
)(_embed_body)


def kernel(x, token_embd, pos_embd):
    return _embed(x.astype(jnp.int32).reshape(B * S), token_embd, pos_embd)


# speedup vs baseline: 1.3726x; 1.1717x over previous
import functools

import jax
import jax.numpy as jnp
from jax import lax
from jax.experimental import pallas as pl
from jax.experimental.pallas import tpu as pltpu
from jax.experimental.pallas import tpu_sc as plsc

B, S, D = 16, 2048, 128
L = 16
BP = 2

_info = plsc.get_sparse_core_info()
NC, NS = _info.num_cores, _info.num_subcores
NW = NC * NS
SW = S // NW
NG = B // BP


def _embed_body(x_hbm, tok_hbm, pos_hbm, out_hbm,
                idx_v, pos_v, buf0, buf1, g0, g1, o0, o1):
    wid = lax.axis_index("s") * NC + lax.axis_index("c")
    s0 = wid * SW

    stage = [
        pltpu.async_copy(x_hbm.at[pl.ds(b * S + s0, SW)],
                         idx_v.at[pl.ds(b * SW, SW)], o0)
        for b in range(B)
    ]
    stage.append(pltpu.async_copy(pos_hbm.at[pl.ds(s0, SW)], pos_v, o0))
    for cp in stage:
        cp.wait()

    bufs = (buf0, buf1)
    gsems = (g0, g1)
    osems = (o0, o1)

    def start_gather(k):
        return pltpu.async_copy(tok_hbm.at[idx_v.at[pl.ds(k * BP * SW, BP * SW)]],
                                bufs[k % 2], gsems[k % 2])

    gcp = start_gather(0)
    ocp = [[], []]
    for k in range(NG):
        cur = k % 2
        buf = bufs[cur]
        this_g = gcp
        if k + 1 < NG:
            nxt = (k + 1) % 2
            for cp in ocp[nxt]:
                cp.wait()
            ocp[nxt] = []
            gcp = start_gather(k + 1)
        this_g.wait()

        def add_row(r, carry, buf=buf):
            for c in range(D // L):
                sl = pl.ds(c * L, L)
                p = pos_v[r, sl]
                for h in range(BP):
                    buf[h * SW + r, sl] = buf[h * SW + r, sl] + p
            return carry

        lax.fori_loop(0, SW, add_row, 0)

        ocp[cur] = [
            pltpu.async_copy(buf.at[pl.ds(h * SW, SW)],
                             out_hbm.at[k * BP + h, pl.ds(s0, SW)], osems[cur])
            for h in range(BP)
        ]

    for cps in ocp:
        for cp in cps:
            cp.wait()


_embed = functools.partial(
    pl.kernel,
    out_type=jax.ShapeDtypeStruct((B, S, D), jnp.float32),
    mesh=plsc.VectorSubcoreMesh(core_axis_name="c", subcore_axis_name="s"),
    scratch_types=[
        pltpu.VMEM((B * SW,), jnp.int32),
        pltpu.VMEM((SW, D), jnp.float32),
        pltpu.VMEM((BP * SW, D), jnp.float32),
        pltpu.VMEM((BP * SW, D), jnp.float32),
        pltpu.SemaphoreType.DMA,
        pltpu.SemaphoreType.DMA,
        pltpu.SemaphoreType.DMA,
        pltpu.SemaphoreType.DMA,
    ],
)(_embed_body)


def kernel(x, token_embd, pos_embd):
    return _embed(x.astype(jnp.int32).reshape(B * S), token_embd, pos_embd)


# speedup vs baseline: 1.3829x; 1.0075x over previous
import functools

import jax
import jax.numpy as jnp
from jax import lax
from jax.experimental import pallas as pl
from jax.experimental.pallas import tpu as pltpu
from jax.experimental.pallas import tpu_sc as plsc

B, S, D = 16, 2048, 128
L = 16
BP = 2
BA = 4

_info = plsc.get_sparse_core_info()
NC, NS = _info.num_cores, _info.num_subcores
NW = NC * NS
SW = S // NW
NG = B // BP
NA = B // BA
NBUF = 4


def _embed_body(x_hbm, tok_hbm, pos_hbm, out_hbm,
                idx2, idxf, pos_v, bufs, gsems, osems):
    wid = lax.axis_index("s") * NC + lax.axis_index("c")
    s0 = wid * SW
    blk = pl.multiple_of((wid // 2) * (2 * SW), 2 * SW)
    h0 = pl.multiple_of((wid % 2) * SW, SW)

    st_idx = pltpu.async_copy(x_hbm.at[:, pl.ds(blk, 2 * SW)], idx2, gsems[0])
    st_pos = pltpu.async_copy(pos_hbm.at[pl.ds(s0, SW)], pos_v, gsems[1])
    st_idx.wait()
    for b in range(B):
        for j in range(SW // L):
            idxf[pl.ds(b * SW + j * L, L)] = idx2[b, pl.ds(h0 + j * L, L)]
    st_pos.wait()

    def start_gather(k):
        return pltpu.async_copy(
            tok_hbm.at[idxf.at[pl.ds(k * BP * SW, BP * SW)]],
            bufs[k % NBUF], gsems[k % NBUF])

    gcp = {0: start_gather(0), 1: start_gather(1)}
    ocp = [[] for _ in range(NBUF)]
    for a in range(NA):
        k0, k1 = 2 * a, 2 * a + 1
        bufP, bufQ = bufs[k0 % NBUF], bufs[k1 % NBUF]
        if a + 1 < NA:
            for kn in (k0 + 2, k1 + 2):
                for cp in ocp[kn % NBUF]:
                    cp.wait()
                ocp[kn % NBUF] = []
                gcp[kn] = start_gather(kn)
        gcp.pop(k0).wait()
        gcp.pop(k1).wait()

        def add_row(r, carry, bufP=bufP, bufQ=bufQ):
            for c in range(D // L):
                sl = pl.ds(c * L, L)
                p = pos_v[r, sl]
                bufP[r, sl] = bufP[r, sl] + p
                bufP[SW + r, sl] = bufP[SW + r, sl] + p
                bufQ[r, sl] = bufQ[r, sl] + p
                bufQ[SW + r, sl] = bufQ[SW + r, sl] + p
            return carry

        lax.fori_loop(0, SW, add_row, 0)

        for k, buf in ((k0, bufP), (k1, bufQ)):
            ocp[k % NBUF] = [
                pltpu.async_copy(buf.at[pl.ds(h * SW, SW)],
                                 out_hbm.at[k * BP + h, pl.ds(s0, SW)],
                                 osems[k % NBUF])
                for h in range(BP)
            ]

    for cps in ocp:
        for cp in cps:
            cp.wait()


def _embed_entry(x_hbm, tok_hbm, pos_hbm, out_hbm,
                 idx2, idxf, pos_v, b0, b1, b2, b3,
                 g0, g1, g2, g3, o0, o1, o2, o3):
    _embed_body(x_hbm, tok_hbm, pos_hbm, out_hbm, idx2, idxf, pos_v,
                (b0, b1, b2, b3), (g0, g1, g2, g3), (o0, o1, o2, o3))


_embed = functools.partial(
    pl.kernel,
    out_type=jax.ShapeDtypeStruct((B, S, D), jnp.float32),
    mesh=plsc.VectorSubcoreMesh(core_axis_name="c", subcore_axis_name="s"),
    scratch_types=(
        [pltpu.VMEM((B, 2 * SW), jnp.int32),
         pltpu.VMEM((B * SW,), jnp.int32),
         pltpu.VMEM((SW, D), jnp.float32)]
        + [pltpu.VMEM((BP * SW, D), jnp.float32)] * NBUF
        + [pltpu.SemaphoreType.DMA] * (2 * NBUF)
    ),
)(_embed_entry)


def kernel(x, token_embd, pos_embd):
    return _embed(x.astype(jnp.int32), token_embd, pos_embd)


# speedup vs baseline: 1.3895x; 1.0048x over previous
import functools

import jax
import jax.numpy as jnp
from jax import lax
from jax.experimental import pallas as pl
from jax.experimental.pallas import tpu as pltpu
from jax.experimental.pallas import tpu_sc as plsc

B, S, D = 16, 2048, 128
L = 16
BP = 2
BA = 4

_info = plsc.get_sparse_core_info()
NC, NS = _info.num_cores, _info.num_subcores
NW = NC * NS
SW = S // NW
NG = B // BP
NA = B // BA
NBUF = 4


def _embed_body(x_hbm, tok_hbm, pos_hbm, out_hbm,
                idx2, idxf, pos_v, bufs, gsems, osems):
    wid = lax.axis_index("s") * NC + lax.axis_index("c")
    s0 = wid * SW
    blk = pl.multiple_of((wid // 2) * (2 * SW), 2 * SW)
    h0 = pl.multiple_of((wid % 2) * SW, SW)

    st_idx = pltpu.async_copy(x_hbm.at[:, pl.ds(blk, 2 * SW)], idx2, gsems[0])
    st_pos = pltpu.async_copy(pos_hbm.at[pl.ds(s0, SW)], pos_v, gsems[1])
    st_idx.wait()
    for b in range(B):
        for j in range(SW // L):
            idxf[pl.ds(b * SW + j * L, L)] = idx2[b, pl.ds(h0 + j * L, L)]
    st_pos.wait()

    def start_gather(k):
        return pltpu.async_copy(
            tok_hbm.at[idxf.at[pl.ds(k * BP * SW, BP * SW)]],
            bufs[k % NBUF], gsems[k % NBUF])

    gcp = {0: start_gather(0), 1: start_gather(1)}
    ocp = [[] for _ in range(NBUF)]
    for a in range(NA):
        k0, k1 = 2 * a, 2 * a + 1
        bufP, bufQ = bufs[k0 % NBUF], bufs[k1 % NBUF]
        if a + 1 < NA:
            for kn in (k0 + 2, k1 + 2):
                for cp in ocp[kn % NBUF]:
                    cp.wait()
                ocp[kn % NBUF] = []
                gcp[kn] = start_gather(kn)
        gcp.pop(k0).wait()
        gcp.pop(k1).wait()

        def add_row(r, carry, bufP=bufP, bufQ=bufQ):
            for c in range(D // L):
                sl = pl.ds(c * L, L)
                p = pos_v[r, sl]
                plsc.addupdate(bufP.at[r, sl], p)
                plsc.addupdate(bufP.at[SW + r, sl], p)
                plsc.addupdate(bufQ.at[r, sl], p)
                plsc.addupdate(bufQ.at[SW + r, sl], p)
            return carry

        lax.fori_loop(0, SW, add_row, 0)

        for k, buf in ((k0, bufP), (k1, bufQ)):
            ocp[k % NBUF] = [
                pltpu.async_copy(buf.at[pl.ds(h * SW, SW)],
                                 out_hbm.at[k * BP + h, pl.ds(s0, SW)],
                                 osems[k % NBUF])
                for h in range(BP)
            ]

    for cps in ocp:
        for cp in cps:
            cp.wait()


def _embed_entry(x_hbm, tok_hbm, pos_hbm, out_hbm,
                 idx2, idxf, pos_v, b0, b1, b2, b3,
                 g0, g1, g2, g3, o0, o1, o2, o3):
    _embed_body(x_hbm, tok_hbm, pos_hbm, out_hbm, idx2, idxf, pos_v,
                (b0, b1, b2, b3), (g0, g1, g2, g3), (o0, o1, o2, o3))


_embed = functools.partial(
    pl.kernel,
    out_type=jax.ShapeDtypeStruct((B, S, D), jnp.float32),
    mesh=plsc.VectorSubcoreMesh(core_axis_name="c", subcore_axis_name="s"),
    scratch_types=(
        [pltpu.VMEM((B, 2 * SW), jnp.int32),
         pltpu.VMEM((B * SW,), jnp.int32),
         pltpu.VMEM((SW, D), jnp.float32)]
        + [pltpu.VMEM((BP * SW, D), jnp.float32)] * NBUF
        + [pltpu.SemaphoreType.DMA] * (2 * NBUF)
    ),
)(_embed_entry)


def kernel(x, token_embd, pos_embd):
    return _embed(x.astype(jnp.int32), token_embd, pos_embd)


# speedup vs baseline: 1.4421x; 1.0378x over previous
import functools

import jax
import jax.numpy as jnp
from jax import lax
from jax.experimental import pallas as pl
from jax.experimental.pallas import tpu as pltpu
from jax.experimental.pallas import tpu_sc as plsc

B, S, D = 16, 2048, 128
L = 16
BP = 2
BA = 4

_info = plsc.get_sparse_core_info()
NC, NS = _info.num_cores, _info.num_subcores
NW = NC * NS
SW = S // NW
NG = B // BP
NA = B // BA
NBUF = 4


def _embed_body(x_hbm, tok_hbm, pos_hbm, out_hbm,
                idx2, idxf, pos_v, bufs, gsems, osems):
    wid = lax.axis_index("s") * NC + lax.axis_index("c")
    s0 = wid * SW
    blk = pl.multiple_of((wid // 2) * (2 * SW), 2 * SW)
    h0 = pl.multiple_of((wid % 2) * SW, SW)

    st_idx = pltpu.async_copy(x_hbm.at[:, pl.ds(blk, 2 * SW)], idx2, gsems[0])
    st_pos = pltpu.async_copy(pos_hbm.at[pl.ds(s0, SW)], pos_v, gsems[1])
    st_idx.wait()
    for b in range(B):
        for j in range(SW // L):
            idxf[pl.ds(b * SW + j * L, L)] = idx2[b, pl.ds(h0 + j * L, L)]
    st_pos.wait()

    def start_gather(k):
        return pltpu.async_copy(
            tok_hbm.at[idxf.at[pl.ds(k * BP * SW, BP * SW)]],
            bufs[k % NBUF], gsems[k % NBUF])

    gcp = {0: start_gather(0), 1: start_gather(1)}
    ocp = [[] for _ in range(NBUF)]
    for k in range(NG):
        buf = bufs[k % NBUF]
        if k + 2 < NG:
            for cp in ocp[(k + 2) % NBUF]:
                cp.wait()
            ocp[(k + 2) % NBUF] = []
            gcp[k + 2] = start_gather(k + 2)
        gcp.pop(k).wait()

        def add_row(r, carry, buf=buf):
            for c in range(D // L):
                sl = pl.ds(c * L, L)
                p = pos_v[r, sl]
                plsc.addupdate(buf.at[r, sl], p)
                plsc.addupdate(buf.at[SW + r, sl], p)
            return carry

        lax.fori_loop(0, SW, add_row, 0)

        ocp[k % NBUF] = [
            pltpu.async_copy(buf.at[pl.ds(h * SW, SW)],
                             out_hbm.at[k * BP + h, pl.ds(s0, SW)],
                             osems[k % NBUF])
            for h in range(BP)
        ]

    for cps in ocp:
        for cp in cps:
            cp.wait()


def _embed_entry(x_hbm, tok_hbm, pos_hbm, out_hbm,
                 idx2, idxf, pos_v, b0, b1, b2, b3,
                 g0, g1, g2, g3, o0, o1, o2, o3):
    _embed_body(x_hbm, tok_hbm, pos_hbm, out_hbm, idx2, idxf, pos_v,
                (b0, b1, b2, b3), (g0, g1, g2, g3), (o0, o1, o2, o3))


_embed = functools.partial(
    pl.kernel,
    out_type=jax.ShapeDtypeStruct((B, S, D), jnp.float32),
    mesh=plsc.VectorSubcoreMesh(core_axis_name="c", subcore_axis_name="s"),
    scratch_types=(
        [pltpu.VMEM((B, 2 * SW), jnp.int32),
         pltpu.VMEM((B * SW,), jnp.int32),
         pltpu.VMEM((SW, D), jnp.float32)]
        + [pltpu.VMEM((BP * SW, D), jnp.float32)] * NBUF
        + [pltpu.SemaphoreType.DMA] * (2 * NBUF)
    ),
)(_embed_entry)


def kernel(x, token_embd, pos_embd):
    return _embed(x.astype(jnp.int32), token_embd, pos_embd)


# speedup vs baseline: 1.4824x; 1.0280x over previous
import functools

import jax
import jax.numpy as jnp
from jax import lax
from jax.experimental import pallas as pl
from jax.experimental.pallas import tpu as pltpu
from jax.experimental.pallas import tpu_sc as plsc

B, S, D = 16, 2048, 128
L = 16
BP = 2
BA = 4

_info = plsc.get_sparse_core_info()
NC, NS = _info.num_cores, _info.num_subcores
NW = NC * NS
SW = S // NW
NG = B // BP
NA = B // BA
NBUF = 5


def _embed_body(x_hbm, tok_hbm, pos_hbm, out_hbm,
                idx2, idxf, pos_v, bufs, gsems, osems):
    wid = lax.axis_index("s") * NC + lax.axis_index("c")
    s0 = wid * SW
    blk = pl.multiple_of((wid // 2) * (2 * SW), 2 * SW)
    h0 = pl.multiple_of((wid % 2) * SW, SW)

    st_idx = pltpu.async_copy(x_hbm.at[:, pl.ds(blk, 2 * SW)], idx2, gsems[0])
    st_pos = pltpu.async_copy(pos_hbm.at[pl.ds(s0, SW)], pos_v, gsems[1])
    st_idx.wait()

    def start_gather(k):
        return pltpu.async_copy(
            tok_hbm.at[idxf.at[pl.ds(k * BP * SW, BP * SW)]],
            bufs[k % NBUF], gsems[k % NBUF])

    def rearrange(b):
        for j in range(SW // L):
            idxf[pl.ds(b * SW + j * L, L)] = idx2[b, pl.ds(h0 + j * L, L)]

    for b in range(BP):
        rearrange(b)
    gcp = {0: start_gather(0)}
    for b in range(BP, B):
        rearrange(b)
    gcp[1] = start_gather(1)
    gcp[2] = start_gather(2)
    st_pos.wait()

    ocp = [[] for _ in range(NBUF)]
    for k in range(NG):
        buf = bufs[k % NBUF]
        if k + 3 < NG:
            for cp in ocp[(k + 3) % NBUF]:
                cp.wait()
            ocp[(k + 3) % NBUF] = []
            gcp[k + 3] = start_gather(k + 3)
        gcp.pop(k).wait()

        def add_row(r, carry, buf=buf):
            for c in range(D // L):
                sl = pl.ds(c * L, L)
                p = pos_v[r, sl]
                plsc.addupdate(buf.at[r, sl], p)
                plsc.addupdate(buf.at[SW + r, sl], p)
            return carry

        lax.fori_loop(0, SW, add_row, 0)

        ocp[k % NBUF] = [
            pltpu.async_copy(buf.at[pl.ds(h * SW, SW)],
                             out_hbm.at[k * BP + h, pl.ds(s0, SW)],
                             osems[k % NBUF])
            for h in range(BP)
        ]

    for cps in ocp:
        for cp in cps:
            cp.wait()


def _embed_entry(x_hbm, tok_hbm, pos_hbm, out_hbm,
                 idx2, idxf, pos_v, b0, b1, b2, b3, b4,
                 g0, g1, g2, g3, g4, o0, o1, o2, o3, o4):
    _embed_body(x_hbm, tok_hbm, pos_hbm, out_hbm, idx2, idxf, pos_v,
                (b0, b1, b2, b3, b4), (g0, g1, g2, g3, g4),
                (o0, o1, o2, o3, o4))


_embed = functools.partial(
    pl.kernel,
    out_type=jax.ShapeDtypeStruct((B, S, D), jnp.float32),
    mesh=plsc.VectorSubcoreMesh(core_axis_name="c", subcore_axis_name="s"),
    scratch_types=(
        [pltpu.VMEM((B, 2 * SW), jnp.int32),
         pltpu.VMEM((B * SW,), jnp.int32),
         pltpu.VMEM((SW, D), jnp.float32)]
        + [pltpu.VMEM((BP * SW, D), jnp.float32)] * NBUF
        + [pltpu.SemaphoreType.DMA] * (2 * NBUF)
    ),
)(_embed_entry)


def kernel(x, token_embd, pos_embd):
    return _embed(x.astype(jnp.int32), token_embd, pos_embd)


# speedup vs baseline: 1.4867x; 1.0029x over previous
import functools

import jax
import jax.numpy as jnp
from jax import lax
from jax.experimental import pallas as pl
from jax.experimental.pallas import tpu as pltpu
from jax.experimental.pallas import tpu_sc as plsc

B, S, D = 16, 2048, 128
L = 16
BP = 2

_info = plsc.get_sparse_core_info()
NC, NS = _info.num_cores, _info.num_subcores
NW = NC * NS
SW = S // NW
NG = B // BP
NBUF = 5


def _embed_body(x_hbm, tok_hbm, pos_hbm, out_hbm,
                idx2, idxf, pos_v, bufs, gsems, osems):
    wid = lax.axis_index("s") * NC + lax.axis_index("c")
    s0 = wid * SW
    blk = pl.multiple_of((wid // 2) * (2 * SW), 2 * SW)
    h0 = pl.multiple_of((wid % 2) * SW, SW)

    st_idx = pltpu.async_copy(x_hbm.at[:, pl.ds(blk, 2 * SW)], idx2, gsems[0])
    st_pos = pltpu.async_copy(pos_hbm.at[pl.ds(s0, SW)], pos_v, gsems[1])
    st_idx.wait()

    def start_gather(k):
        return pltpu.async_copy(
            tok_hbm.at[idxf.at[pl.ds(k * BP * SW, BP * SW)]],
            bufs[k % NBUF], gsems[k % NBUF])

    def rearrange(b):
        for j in range(SW // L):
            idxf[pl.ds(b * SW + j * L, L)] = idx2[b, pl.ds(h0 + j * L, L)]

    for b in range(BP):
        rearrange(b)
    gcp = {0: start_gather(0)}
    for b in range(BP, B):
        rearrange(b)
    gcp[1] = start_gather(1)
    gcp[2] = start_gather(2)
    st_pos.wait()

    ocp = [[] for _ in range(NBUF)]
    for k in range(NG):
        buf = bufs[k % NBUF]
        if k + 3 < NG:
            for cp in ocp[(k + 3) % NBUF]:
                cp.wait()
            ocp[(k + 3) % NBUF] = []
            gcp[k + 3] = start_gather(k + 3)
        gcp.pop(k).wait()

        def add_row(r, carry, buf=buf):
            for c in range(D // L):
                sl = pl.ds(c * L, L)
                p = pos_v[r, sl]
                plsc.addupdate(buf.at[r, sl], p)
                plsc.addupdate(buf.at[SW + r, sl], p)
            return carry

        lax.fori_loop(0, SW, add_row, 0)

        ocp[k % NBUF] = [
            pltpu.async_copy(buf.at[pl.ds(h * SW, SW)],
                             out_hbm.at[k * BP + h, pl.ds(s0, SW)],
                             osems[k % NBUF])
            for h in range(BP)
        ]

    for cps in ocp:
        for cp in cps:
            cp.wait()


def _embed_entry(x_hbm, tok_hbm, pos_hbm, out_hbm,
                 idx2, idxf, pos_v, b0, b1, b2, b3, b4,
                 g0, g1, g2, g3, g4, o0, o1, o2, o3, o4):
    _embed_body(x_hbm, tok_hbm, pos_hbm, out_hbm, idx2, idxf, pos_v,
                (b0, b1, b2, b3, b4), (g0, g1, g2, g3, g4),
                (o0, o1, o2, o3, o4))


_embed = functools.partial(
    pl.kernel,
    out_type=jax.ShapeDtypeStruct((B, S, D), jnp.float32),
    mesh=plsc.VectorSubcoreMesh(core_axis_name="c", subcore_axis_name="s"),
    scratch_types=(
        [pltpu.VMEM((B, 2 * SW), jnp.int32),
         pltpu.VMEM((B * SW,), jnp.int32),
         pltpu.VMEM((SW, D), jnp.float32)]
        + [pltpu.VMEM((BP * SW, D), jnp.float32)] * NBUF
        + [pltpu.SemaphoreType.DMA] * (2 * NBUF)
    ),
)(_embed_entry)


def kernel(x, token_embd, pos_embd):
    return _embed(x.astype(jnp.int32), token_embd, pos_embd)
